# Initial kernel scaffold; baseline (speedup 1.0000x reference)
import functools

import jax
import jax.numpy as jnp
from jax import lax
from jax.experimental import pallas as pl
from jax.experimental.pallas import tpu as pltpu
from jax.experimental.pallas import tpu_sc as plsc

N_NODES = 100000
N_EDGES = 1600000
ND = 16
ED = 16
MD = 32

NC = 2
NS = 16
NW = NC * NS

E_PAD = 1605632
E_ROWS = E_PAD // 128

G_GRP = 49
G_EPW = E_PAD // NW


def _gather_body(comb, colsr, rowsr, gi, gj, idx_v, buf_v, sem):
    c = lax.axis_index("c")
    s = lax.axis_index("s")
    wid = s * NC + c

    def do(idx_hbm, out_hbm):
        def grp(g, _):
            b = wid * G_EPW + g * 1024
            brow = wid * (G_EPW // 128) + g * 8
            pltpu.sync_copy(idx_hbm.at[pl.ds(brow, 8)], idx_v)
            descs = []
            for j in range(8):
                descs.append(
                    pltpu.async_copy(
                        comb.at[idx_v.at[j]],
                        buf_v.at[pl.ds(j * 128, 128)],
                        sem,
                    )
                )
            for d in descs:
                d.wait()
            pltpu.sync_copy(buf_v, out_hbm.at[pl.ds(b, 1024)])
            return 0

        lax.fori_loop(0, G_GRP, grp, 0)

    do(colsr, gi)
    do(rowsr, gj)


@jax.jit
def _sc_gather(comb, cols2, rows2):
    return pl.kernel(
        _gather_body,
        out_type=[
            jax.ShapeDtypeStruct((E_PAD, 2 * ND), jnp.float32),
            jax.ShapeDtypeStruct((E_PAD, 2 * ND), jnp.float32),
        ],
        mesh=plsc.VectorSubcoreMesh(core_axis_name="c", subcore_axis_name="s"),
        scratch_types=[
            pltpu.VMEM((8, 128), jnp.int32),
            pltpu.VMEM((1024, 2 * ND), jnp.float32),
            pltpu.SemaphoreType.DMA,
        ],
    )(comb, cols2, rows2)


S_GRP = 98
S_EPT = E_PAD // NS
ACC_ROWS = N_NODES + 8
NPT = N_NODES // NS
ZB = 250


def _scatter_body(msgs, idxs, out, acc, idx_v, mbuf, zbuf, sem):
    c = lax.axis_index("c")
    s = lax.axis_index("s")

    def zrow(i, _):
        zbuf[i, :] = jnp.zeros((16,), jnp.float32)
        return 0

    lax.fori_loop(0, ZB, zrow, 0)
    r0 = s * NPT

    def zchunk(k, _):
        pltpu.sync_copy(zbuf, acc.at[pl.ds(r0 + k * ZB, ZB)])
        return 0

    lax.fori_loop(0, NPT // ZB, zchunk, 0)

    plsc.subcore_barrier()

    def grp(g, _):
        b = s * S_EPT + g * 1024
        brow = s * (S_EPT // 128) + g * 8
        pltpu.sync_copy(idxs.at[pl.ds(brow, 8)], idx_v)
        pltpu.sync_copy(msgs.at[c, pl.ds(b, 1024)], mbuf)
        for j in range(8):
            pltpu.sync_copy(
                mbuf.at[pl.ds(j * 128, 128)],
                acc.at[idx_v.at[j]],
                add=True,
            )
        return 0

    lax.fori_loop(0, S_GRP, grp, 0)
    plsc.subcore_barrier()
    pltpu.sync_copy(acc.at[pl.ds(r0, NPT)], out.at[c, pl.ds(r0, NPT)])


@jax.jit
def _sc_scatter(msgs, idxs2):
    return pl.kernel(
        _scatter_body,
        out_type=jax.ShapeDtypeStruct((2, N_NODES, ED), jnp.float32),
        mesh=plsc.VectorSubcoreMesh(core_axis_name="c", subcore_axis_name="s"),
        scratch_types=[
            pltpu.VMEM_SHARED((ACC_ROWS, ED), jnp.float32),
            pltpu.VMEM((8, 128), jnp.int32),
            pltpu.VMEM((1024, ED), jnp.float32),
            pltpu.VMEM((ZB, ED), jnp.float32),
            pltpu.SemaphoreType.DMA,
        ],
    )(msgs, idxs2)


BE = 8192


def _edge_body(gi, gj, ea, w0, b0, w1, b1, w2, b2,
               fw0, fb0, fw1, fb1, pw0, pb0, pw1, pb1,
               ue, past2, fut2):
    xi = gi[:, :ND]
    ii = gi[:, ND:]
    xj = gj[:, :ND]
    ij = gj[:, ND:]
    e = ea[...]
    cat = jnp.concatenate([xi, xj, e], axis=1)
    h = jax.nn.relu(jnp.dot(cat, w0[...], preferred_element_type=jnp.float32) + b0[...])
    h = jax.nn.relu(jnp.dot(h, w1[...], preferred_element_type=jnp.float32) + b1[...])
    u = jnp.dot(h, w2[...], preferred_element_type=jnp.float32) + b2[...]
    ue[...] = u
    fcat = jnp.concatenate([xi, u, ii], axis=1)
    fh = jax.nn.relu(jnp.dot(fcat, fw0[...], preferred_element_type=jnp.float32) + fb0[...])
    fm = jnp.dot(fh, fw1[...], preferred_element_type=jnp.float32) + fb1[...]
    fut2[0] = fm[:, :ED]
    fut2[1] = fm[:, ED:]
    pcat = jnp.concatenate([xj, u, ij], axis=1)
    ph = jax.nn.relu(jnp.dot(pcat, pw0[...], preferred_element_type=jnp.float32) + pb0[...])
    pm = jnp.dot(ph, pw1[...], preferred_element_type=jnp.float32) + pb1[...]
    past2[0] = pm[:, :ED]
    past2[1] = pm[:, ED:]


def _wspec(shape):
    return pl.BlockSpec(shape, lambda i: (0,) * len(shape))


@jax.jit
def _tc_edge(gi, gj, ea, w0, b0, w1, b1, w2, b2,
             fw0, fb0, fw1, fb1, pw0, pb0, pw1, pb1):
    grid = E_PAD // BE
    return pl.pallas_call(
        _edge_body,
        grid=(grid,),
        in_specs=[
            pl.BlockSpec((BE, 2 * ND), lambda i: (i, 0)),
            pl.BlockSpec((BE, 2 * ND), lambda i: (i, 0)),
            pl.BlockSpec((BE, ED), lambda i: (i, 0)),
            _wspec(w0.shape), _wspec(b0.shape),
            _wspec(w1.shape), _wspec(b1.shape),
            _wspec(w2.shape), _wspec(b2.shape),
            _wspec(fw0.shape), _wspec(fb0.shape),
            _wspec(fw1.shape), _wspec(fb1.shape),
            _wspec(pw0.shape), _wspec(pb0.shape),
            _wspec(pw1.shape), _wspec(pb1.shape),
        ],
        out_specs=[
            pl.BlockSpec((BE, ED), lambda i: (i, 0)),
            pl.BlockSpec((2, BE, ED), lambda i: (0, i, 0)),
            pl.BlockSpec((2, BE, ED), lambda i: (0, i, 0)),
        ],
        out_shape=[
            jax.ShapeDtypeStruct((N_EDGES, ED), jnp.float32),
            jax.ShapeDtypeStruct((2, E_PAD, ED), jnp.float32),
            jax.ShapeDtypeStruct((2, E_PAD, ED), jnp.float32),
        ],
    )(gi, gj, ea, w0, b0, w1, b1, w2, b2,
      fw0, fb0, fw1, fb1, pw0, pb0, pw1, pb1)


BN = 8192


def _node_body(mp, mf, w0, b0, w1, b1, out):
    m = jnp.concatenate([mp[0], mp[1], mf[0], mf[1]], axis=1)
    h = jax.nn.relu(jnp.dot(m, w0[...], preferred_element_type=jnp.float32) + b0[...])
    out[...] = jnp.dot(h, w1[...], preferred_element_type=jnp.float32) + b1[...]


@jax.jit
def _tc_node(mp, mf, w0, b0, w1, b1):
    grid = pl.cdiv(N_NODES, BN)
    return pl.pallas_call(
        _node_body,
        grid=(grid,),
        in_specs=[
            pl.BlockSpec((2, BN, ED), lambda i: (0, i, 0)),
            pl.BlockSpec((2, BN, ED), lambda i: (0, i, 0)),
            _wspec(w0.shape), _wspec(b0.shape),
            _wspec(w1.shape), _wspec(b1.shape),
        ],
        out_specs=pl.BlockSpec((BN, ND), lambda i: (i, 0)),
        out_shape=jax.ShapeDtypeStruct((N_NODES, ND), jnp.float32),
    )(mp, mf, w0, b0, w1, b1)


def kernel(x, edge_index, edge_attr, initial_x,
           eu_w0, eu_b0, eu_w1, eu_b1, eu_w2, eu_b2,
           pm_w0, pm_b0, pm_w1, pm_b1,
           fm_w0, fm_b0, fm_w1, fm_b1,
           cf_w0, cf_b0, cf_w1, cf_b1):
    rows = edge_index[0]
    cols = edge_index[1]
    pad = E_PAD - N_EDGES
    zpad = jnp.zeros((pad,), jnp.int32)
    tpad = jnp.full((pad,), N_NODES, jnp.int32)
    cols_g = jnp.concatenate([cols, zpad]).reshape(E_ROWS, 128)
    rows_g = jnp.concatenate([rows, zpad]).reshape(E_ROWS, 128)
    cols_s = jnp.concatenate([cols, tpad]).reshape(E_ROWS, 128)
    rows_s = jnp.concatenate([rows, tpad]).reshape(E_ROWS, 128)

    eu_b0r = eu_b0.reshape(1, -1)
    eu_b1r = eu_b1.reshape(1, -1)
    eu_b2r = eu_b2.reshape(1, -1)
    pm_b0r = pm_b0.reshape(1, -1)
    pm_b1r = pm_b1.reshape(1, -1)
    fm_b0r = fm_b0.reshape(1, -1)
    fm_b1r = fm_b1.reshape(1, -1)
    cf_b0r = cf_b0.reshape(1, -1)
    cf_b1r = cf_b1.reshape(1, -1)

    for _ in range(3):
        comb = jnp.concatenate([x, initial_x], axis=1)
        gi, gj = _sc_gather(comb, cols_g, rows_g)
        ue, past2, fut2 = _tc_edge(
            gi, gj, edge_attr,
            eu_w0, eu_b0r, eu_w1, eu_b1r, eu_w2, eu_b2r,
            fm_w0, fm_b0r, fm_w1, fm_b1r,
            pm_w0, pm_b0r, pm_w1, pm_b1r,
        )
        mp = _sc_scatter(past2, cols_s)
        mf = _sc_scatter(fut2, rows_s)
        x = _tc_node(mp, mf, cf_w0, cf_b0r, cf_w1, cf_b1r)
        edge_attr = ue
    return x, edge_attr


# speedup vs baseline: 5.5196x; 5.5196x over previous
import functools

import jax
import jax.numpy as jnp
from jax import lax
from jax.experimental import pallas as pl
from jax.experimental.pallas import tpu as pltpu
from jax.experimental.pallas import tpu_sc as plsc

N_NODES = 100000
N_EDGES = 1600000
ND = 16
ED = 16
MD = 32

NC = 2
NS = 16
NW = NC * NS

E_PAD = 1605632
E_ROWS = E_PAD // 128

G_GRP = 49
G_EPW = E_PAD // NW


def _gather_body(comb, colsr, rowsr, gi, gj, idx_v, buf_v, sem):
    c = lax.axis_index("c")
    s = lax.axis_index("s")
    wid = s * NC + c

    def do(idx_hbm, out_hbm):
        def grp(g, _):
            b = wid * G_EPW + g * 1024
            brow = wid * (G_EPW // 128) + g * 8
            pltpu.sync_copy(idx_hbm.at[pl.ds(brow, 8)], idx_v)
            descs = []
            for j in range(8):
                descs.append(
                    pltpu.async_copy(
                        comb.at[idx_v.at[j]],
                        buf_v.at[pl.ds(j * 128, 128)],
                        sem,
                    )
                )
            for d in descs:
                d.wait()
            pltpu.sync_copy(buf_v, out_hbm.at[pl.ds(b, 1024)])
            return 0

        lax.fori_loop(0, G_GRP, grp, 0)

    do(colsr, gi)
    do(rowsr, gj)


@jax.jit
def _sc_gather(comb, cols2, rows2):
    return pl.kernel(
        _gather_body,
        out_type=[
            jax.ShapeDtypeStruct((E_PAD, 2 * ND), jnp.float32),
            jax.ShapeDtypeStruct((E_PAD, 2 * ND), jnp.float32),
        ],
        mesh=plsc.VectorSubcoreMesh(core_axis_name="c", subcore_axis_name="s"),
        scratch_types=[
            pltpu.VMEM((8, 128), jnp.int32),
            pltpu.VMEM((1024, 2 * ND), jnp.float32),
            pltpu.SemaphoreType.DMA,
        ],
        compiler_params=pltpu.CompilerParams(use_tc_tiling_on_sc=False),
    )(comb, cols2, rows2)


S_GRP = 98
S_EPT = E_PAD // NS
ACC_ROWS = N_NODES + 8
NPT = N_NODES // NS
ZB = 250


def _scatter_body(msgs, idxs, out, acc, idx_v, mbuf, zbuf, sem):
    c = lax.axis_index("c")
    s = lax.axis_index("s")

    def zrow(i, _):
        zbuf[i, :] = jnp.zeros((16,), jnp.float32)
        return 0

    lax.fori_loop(0, ZB, zrow, 0)
    r0 = s * NPT

    def zchunk(k, _):
        pltpu.sync_copy(zbuf, acc.at[pl.ds(r0 + k * ZB, ZB)])
        return 0

    lax.fori_loop(0, NPT // ZB, zchunk, 0)

    plsc.subcore_barrier()

    def grp(g, _):
        b = s * S_EPT + g * 1024
        brow = s * (S_EPT // 128) + g * 8
        pltpu.sync_copy(idxs.at[pl.ds(brow, 8)], idx_v)
        pltpu.sync_copy(msgs.at[c, pl.ds(b, 1024)], mbuf)
        for j in range(8):
            pltpu.sync_copy(
                mbuf.at[pl.ds(j * 128, 128)],
                acc.at[idx_v.at[j]],
                add=True,
            )
        return 0

    lax.fori_loop(0, S_GRP, grp, 0)
    plsc.subcore_barrier()
    pltpu.sync_copy(acc.at[pl.ds(r0, NPT)], out.at[c, pl.ds(r0, NPT)])


@jax.jit
def _sc_scatter(msgs, idxs2):
    return pl.kernel(
        _scatter_body,
        out_type=jax.ShapeDtypeStruct((2, N_NODES, ED), jnp.float32),
        mesh=plsc.VectorSubcoreMesh(core_axis_name="c", subcore_axis_name="s"),
        scratch_types=[
            pltpu.VMEM_SHARED((ACC_ROWS, ED), jnp.float32),
            pltpu.VMEM((8, 128), jnp.int32),
            pltpu.VMEM((1024, ED), jnp.float32),
            pltpu.VMEM((ZB, ED), jnp.float32),
            pltpu.SemaphoreType.DMA,
        ],
        compiler_params=pltpu.CompilerParams(use_tc_tiling_on_sc=False),
    )(msgs, idxs2)


BR = 512
E_PROWS = E_PAD // 8
E_UROWS = N_EDGES // 8


def _edge_body(gi, gj, ea, wgi, wgj, wea, b0, w1, b1, w2, b2,
               fwg, fwu, fb0, fw1a, fb1a, fw1b, fb1b,
               pwg, pwu, pb0, pw1a, pb1a, pw1b, pb1b,
               ue, past2, fut2):
    f32 = jnp.float32
    G = gi[...]
    J = gj[...]
    h = jax.nn.relu(
        jnp.dot(G, wgi[...], preferred_element_type=f32)
        + jnp.dot(J, wgj[...], preferred_element_type=f32)
        + jnp.dot(ea[...], wea[...], preferred_element_type=f32)
        + b0[...])
    h = jax.nn.relu(jnp.dot(h, w1[...], preferred_element_type=f32) + b1[...])
    u = jnp.dot(h, w2[...], preferred_element_type=f32) + b2[...]
    ue[...] = u
    fh = jax.nn.relu(
        jnp.dot(G, fwg[...], preferred_element_type=f32)
        + jnp.dot(u, fwu[...], preferred_element_type=f32)
        + fb0[...])
    fut2[0] = jnp.dot(fh, fw1a[...], preferred_element_type=f32) + fb1a[...]
    fut2[1] = jnp.dot(fh, fw1b[...], preferred_element_type=f32) + fb1b[...]
    ph = jax.nn.relu(
        jnp.dot(J, pwg[...], preferred_element_type=f32)
        + jnp.dot(u, pwu[...], preferred_element_type=f32)
        + pb0[...])
    past2[0] = jnp.dot(ph, pw1a[...], preferred_element_type=f32) + pb1a[...]
    past2[1] = jnp.dot(ph, pw1b[...], preferred_element_type=f32) + pb1b[...]


def _wspec(shape):
    return pl.BlockSpec(shape, lambda i: (0,) * len(shape))


@jax.jit
def _tc_edge(gi, gj, ea, *ws):
    grid = E_PROWS // BR
    in_specs = [
        pl.BlockSpec((BR, 256), lambda i: (i, 0)),
        pl.BlockSpec((BR, 256), lambda i: (i, 0)),
        pl.BlockSpec((BR, 128), lambda i: (i, 0)),
    ] + [_wspec(w.shape) for w in ws]
    return pl.pallas_call(
        _edge_body,
        grid=(grid,),
        in_specs=in_specs,
        out_specs=[
            pl.BlockSpec((BR, 128), lambda i: (i, 0)),
            pl.BlockSpec((2, BR, 128), lambda i: (0, i, 0)),
            pl.BlockSpec((2, BR, 128), lambda i: (0, i, 0)),
        ],
        out_shape=[
            jax.ShapeDtypeStruct((E_PROWS, 128), jnp.float32),
            jax.ShapeDtypeStruct((2, E_PROWS, 128), jnp.float32),
            jax.ShapeDtypeStruct((2, E_PROWS, 128), jnp.float32),
        ],
    )(gi, gj, ea, *ws)


BRN = 1024
N_PROWS = N_NODES // 8


def _node_body(mp, mf, wp0, wp1, wf0, wf1, b0, w1, b1, out):
    f32 = jnp.float32
    h = jax.nn.relu(
        jnp.dot(mp[0], wp0[...], preferred_element_type=f32)
        + jnp.dot(mp[1], wp1[...], preferred_element_type=f32)
        + jnp.dot(mf[0], wf0[...], preferred_element_type=f32)
        + jnp.dot(mf[1], wf1[...], preferred_element_type=f32)
        + b0[...])
    out[...] = jnp.dot(h, w1[...], preferred_element_type=f32) + b1[...]


@jax.jit
def _tc_node(mp, mf, *ws):
    grid = pl.cdiv(N_PROWS, BRN)
    in_specs = [
        pl.BlockSpec((2, BRN, 128), lambda i: (0, i, 0)),
        pl.BlockSpec((2, BRN, 128), lambda i: (0, i, 0)),
    ] + [_wspec(w.shape) for w in ws]
    return pl.pallas_call(
        _node_body,
        grid=(grid,),
        in_specs=in_specs,
        out_specs=pl.BlockSpec((BRN, 128), lambda i: (i, 0)),
        out_shape=jax.ShapeDtypeStruct((N_PROWS, 128), jnp.float32),
    )(mp, mf, *ws)


def _bd8(w):
    return jnp.kron(jnp.eye(8, dtype=jnp.float32), w)


def _bt8(b):
    return jnp.tile(b, 8).reshape(1, -1)


def kernel(x, edge_index, edge_attr, initial_x,
           eu_w0, eu_b0, eu_w1, eu_b1, eu_w2, eu_b2,
           pm_w0, pm_b0, pm_w1, pm_b1,
           fm_w0, fm_b0, fm_w1, fm_b1,
           cf_w0, cf_b0, cf_w1, cf_b1):
    rows = edge_index[0]
    cols = edge_index[1]
    pad = E_PAD - N_EDGES
    zpad = jnp.zeros((pad,), jnp.int32)
    tpad = jnp.full((pad,), N_NODES, jnp.int32)
    cols_g = jnp.concatenate([cols, zpad]).reshape(E_ROWS, 128)
    rows_g = jnp.concatenate([rows, zpad]).reshape(E_ROWS, 128)
    cols_s = jnp.concatenate([cols, tpad]).reshape(E_ROWS, 128)
    rows_s = jnp.concatenate([rows, tpad]).reshape(E_ROWS, 128)

    z16 = jnp.zeros((ND, 2 * ND), jnp.float32)
    edge_ws = (
        _bd8(jnp.concatenate([eu_w0[:ND], z16], axis=0)),
        _bd8(jnp.concatenate([eu_w0[ND:2 * ND], z16], axis=0)),
        _bd8(eu_w0[2 * ND:]),
        _bt8(eu_b0),
        _bd8(eu_w1), _bt8(eu_b1),
        _bd8(eu_w2), _bt8(eu_b2),
        _bd8(jnp.concatenate([fm_w0[:ND], fm_w0[2 * ND:]], axis=0)),
        _bd8(fm_w0[ND:2 * ND]),
        _bt8(fm_b0),
        _bd8(fm_w1[:, :ED]), _bt8(fm_b1[:ED]),
        _bd8(fm_w1[:, ED:]), _bt8(fm_b1[ED:]),
        _bd8(jnp.concatenate([pm_w0[:ND], pm_w0[2 * ND:]], axis=0)),
        _bd8(pm_w0[ND:2 * ND]),
        _bt8(pm_b0),
        _bd8(pm_w1[:, :ED]), _bt8(pm_b1[:ED]),
        _bd8(pm_w1[:, ED:]), _bt8(pm_b1[ED:]),
    )
    node_ws = (
        _bd8(cf_w0[:ND]), _bd8(cf_w0[ND:2 * ND]),
        _bd8(cf_w0[2 * ND:3 * ND]), _bd8(cf_w0[3 * ND:]),
        _bt8(cf_b0),
        _bd8(cf_w1), _bt8(cf_b1),
    )

    ea_p = jnp.concatenate(
        [edge_attr, jnp.zeros((pad, ED), jnp.float32)]).reshape(E_PROWS, 128)
    for _ in range(3):
        comb = jnp.concatenate([x, initial_x], axis=1)
        gi, gj = _sc_gather(comb, cols_g, rows_g)
        gi_p = gi.reshape(E_PROWS, 256)
        gj_p = gj.reshape(E_PROWS, 256)
        ue_p, past_p, fut_p = _tc_edge(gi_p, gj_p, ea_p, *edge_ws)
        mp = _sc_scatter(past_p.reshape(2, E_PAD, ED), cols_s)
        mf = _sc_scatter(fut_p.reshape(2, E_PAD, ED), rows_s)
        xp = _tc_node(mp.reshape(2, N_PROWS, 128), mf.reshape(2, N_PROWS, 128),
                      *node_ws)
        x = xp.reshape(N_NODES, ND)
        ea_p = ue_p
    return x, ea_p.reshape(E_PAD, ED)[:N_EDGES]


# speedup vs baseline: 6.0452x; 1.0952x over previous
import functools

import jax
import jax.numpy as jnp
from jax import lax
from jax.experimental import pallas as pl
from jax.experimental.pallas import tpu as pltpu
from jax.experimental.pallas import tpu_sc as plsc

N_NODES = 100000
N_EDGES = 1600000
ND = 16
ED = 16
MD = 32

NC = 2
NS = 16
NW = NC * NS

E_PAD = 1605632
E_ROWS = E_PAD // 128

G_GRP = 49
G_EPW = E_PAD // NW


def _gather_body(comb, colsr, rowsr, gi, gj,
                 idx0, idx1, buf0, buf1, si0, si1, sg, sw0, sw1):
    c = lax.axis_index("c")
    s = lax.axis_index("s")
    wid = s * NC + c
    idx_v = (idx0, idx1)
    buf_v = (buf0, buf1)
    si = (si0, si1)
    sw = (sw0, sw1)

    def do(idx_hbm, out_hbm):
        def fire_idx(g, sl):
            brow = wid * (G_EPW // 128) + g * 8
            pltpu.async_copy(idx_hbm.at[pl.ds(brow, 8)], idx_v[sl], si[sl])

        def slot_body(g, sl, k):
            pltpu.make_async_copy(idx_hbm.at[pl.ds(0, 8)], idx_v[sl], si[sl]).wait()

            @pl.when(k > 0)
            def _():
                pltpu.make_async_copy(
                    buf_v[sl], out_hbm.at[pl.ds(0, 1024)], sw[sl]).wait()

            descs = []
            for j in range(8):
                descs.append(pltpu.async_copy(
                    comb.at[idx_v[sl].at[j]],
                    buf_v[sl].at[pl.ds(j * 128, 128)],
                    sg,
                ))
            for d in descs:
                d.wait()
            b = wid * G_EPW + g * 1024
            pltpu.async_copy(buf_v[sl], out_hbm.at[pl.ds(b, 1024)], sw[sl])

            @pl.when(g + 2 < G_GRP)
            def _():
                fire_idx(g + 2, sl)

        fire_idx(0, 0)
        fire_idx(1, 1)

        def pair(k, _):
            slot_body(2 * k, 0, k)
            slot_body(2 * k + 1, 1, k)
            return 0

        lax.fori_loop(0, G_GRP // 2, pair, 0)
        slot_body(G_GRP - 1, 0, 1)
        pltpu.make_async_copy(buf_v[1], out_hbm.at[pl.ds(0, 1024)], sw[1]).wait()
        pltpu.make_async_copy(buf_v[0], out_hbm.at[pl.ds(0, 1024)], sw[0]).wait()

    do(colsr, gi)
    do(rowsr, gj)


@jax.jit
def _sc_gather(comb, cols2, rows2):
    return pl.kernel(
        _gather_body,
        out_type=[
            jax.ShapeDtypeStruct((E_PAD, 2 * ND), jnp.float32),
            jax.ShapeDtypeStruct((E_PAD, 2 * ND), jnp.float32),
        ],
        mesh=plsc.VectorSubcoreMesh(core_axis_name="c", subcore_axis_name="s"),
        scratch_types=[
            pltpu.VMEM((8, 128), jnp.int32),
            pltpu.VMEM((8, 128), jnp.int32),
            pltpu.VMEM((1024, 2 * ND), jnp.float32),
            pltpu.VMEM((1024, 2 * ND), jnp.float32),
            pltpu.SemaphoreType.DMA,
            pltpu.SemaphoreType.DMA,
            pltpu.SemaphoreType.DMA,
            pltpu.SemaphoreType.DMA,
            pltpu.SemaphoreType.DMA,
        ],
        compiler_params=pltpu.CompilerParams(use_tc_tiling_on_sc=False),
    )(comb, cols2, rows2)


S_B = 512
S_NJ = S_B // 128
S_EPT = E_PAD // NS
S_GRP = S_EPT // S_B
ACC_ROWS = N_NODES + 8
NPT = N_NODES // NS
ZB = 125


def _scatter_body(msgs, idxs, out, acc,
                  idxv0, idxv1, mbuf0, mbuf1, zbuf, sd0, sd1, sa0, sa1, sz):
    c = lax.axis_index("c")
    s = lax.axis_index("s")
    idx_v = (idxv0, idxv1)
    mbuf = (mbuf0, mbuf1)
    sd = (sd0, sd1)
    sa = (sa0, sa1)

    def zrow(i, _):
        zbuf[i, :] = jnp.zeros((16,), jnp.float32)
        return 0

    lax.fori_loop(0, ZB, zrow, 0)
    r0 = s * NPT
    for k in range(NPT // ZB):
        pltpu.async_copy(zbuf, acc.at[pl.ds(r0 + k * ZB, ZB)], sz)
    for k in range(NPT // ZB):
        pltpu.make_async_copy(zbuf, acc.at[pl.ds(r0, ZB)], sz).wait()

    plsc.subcore_barrier()

    def fire_loads(g, sl):
        b = s * S_EPT + g * S_B
        brow = s * (S_EPT // 128) + g * S_NJ
        pltpu.async_copy(idxs.at[pl.ds(brow, S_NJ)], idx_v[sl], sd[sl])
        pltpu.async_copy(msgs.at[c, pl.ds(b, S_B)], mbuf[sl], sd[sl])

    def wait_loads(sl):
        pltpu.make_async_copy(idxs.at[pl.ds(0, S_NJ)], idx_v[sl], sd[sl]).wait()
        pltpu.make_async_copy(
            msgs.at[c, pl.ds(0, S_B)], mbuf[sl], sd[sl]).wait()

    def fire_adds(sl):
        descs = []
        for j in range(S_NJ):
            descs.append(pltpu.async_copy(
                mbuf[sl].at[pl.ds(j * 128, 128)],
                acc.at[idx_v[sl].at[j]],
                sa[sl],
                add=True,
            ))
        return descs

    fire_loads(0, 0)
    fire_loads(1, 1)

    def pair(k, _):
        g0 = 2 * k
        wait_loads(0)
        d0 = fire_adds(0)
        wait_loads(1)
        d1 = fire_adds(1)
        for d in d0:
            d.wait()

        @pl.when(g0 + 2 < S_GRP)
        def _():
            fire_loads(g0 + 2, 0)

        for d in d1:
            d.wait()

        @pl.when(g0 + 3 < S_GRP)
        def _():
            fire_loads(g0 + 3, 1)

        return 0

    lax.fori_loop(0, S_GRP // 2, pair, 0)
    plsc.subcore_barrier()
    pltpu.sync_copy(acc.at[pl.ds(r0, NPT)], out.at[c, pl.ds(r0, NPT)])


@jax.jit
def _sc_scatter(msgs, idxs2):
    return pl.kernel(
        _scatter_body,
        out_type=jax.ShapeDtypeStruct((2, N_NODES, ED), jnp.float32),
        mesh=plsc.VectorSubcoreMesh(core_axis_name="c", subcore_axis_name="s"),
        scratch_types=[
            pltpu.VMEM_SHARED((ACC_ROWS, ED), jnp.float32),
            pltpu.VMEM((S_NJ, 128), jnp.int32),
            pltpu.VMEM((S_NJ, 128), jnp.int32),
            pltpu.VMEM((S_B, ED), jnp.float32),
            pltpu.VMEM((S_B, ED), jnp.float32),
            pltpu.VMEM((ZB, ED), jnp.float32),
            pltpu.SemaphoreType.DMA,
            pltpu.SemaphoreType.DMA,
            pltpu.SemaphoreType.DMA,
            pltpu.SemaphoreType.DMA,
            pltpu.SemaphoreType.DMA,
        ],
        compiler_params=pltpu.CompilerParams(use_tc_tiling_on_sc=False),
    )(msgs, idxs2)


BR = 512
E_PROWS = E_PAD // 8
E_UROWS = N_EDGES // 8


def _edge_body(gi, gj, ea, wgi, wgj, wea, b0, w1, b1, w2, b2,
               fwg, fwu, fb0, fw1a, fb1a, fw1b, fb1b,
               pwg, pwu, pb0, pw1a, pb1a, pw1b, pb1b,
               ue, past2, fut2):
    f32 = jnp.float32
    G = gi[...]
    J = gj[...]
    h = jax.nn.relu(
        jnp.dot(G, wgi[...], preferred_element_type=f32)
        + jnp.dot(J, wgj[...], preferred_element_type=f32)
        + jnp.dot(ea[...], wea[...], preferred_element_type=f32)
        + b0[...])
    h = jax.nn.relu(jnp.dot(h, w1[...], preferred_element_type=f32) + b1[...])
    u = jnp.dot(h, w2[...], preferred_element_type=f32) + b2[...]
    ue[...] = u
    fh = jax.nn.relu(
        jnp.dot(G, fwg[...], preferred_element_type=f32)
        + jnp.dot(u, fwu[...], preferred_element_type=f32)
        + fb0[...])
    fut2[0] = jnp.dot(fh, fw1a[...], preferred_element_type=f32) + fb1a[...]
    fut2[1] = jnp.dot(fh, fw1b[...], preferred_element_type=f32) + fb1b[...]
    ph = jax.nn.relu(
        jnp.dot(J, pwg[...], preferred_element_type=f32)
        + jnp.dot(u, pwu[...], preferred_element_type=f32)
        + pb0[...])
    past2[0] = jnp.dot(ph, pw1a[...], preferred_element_type=f32) + pb1a[...]
    past2[1] = jnp.dot(ph, pw1b[...], preferred_element_type=f32) + pb1b[...]


def _wspec(shape):
    return pl.BlockSpec(shape, lambda i: (0,) * len(shape))


@jax.jit
def _tc_edge(gi, gj, ea, *ws):
    grid = E_PROWS // BR
    in_specs = [
        pl.BlockSpec((BR, 256), lambda i: (i, 0)),
        pl.BlockSpec((BR, 256), lambda i: (i, 0)),
        pl.BlockSpec((BR, 128), lambda i: (i, 0)),
    ] + [_wspec(w.shape) for w in ws]
    return pl.pallas_call(
        _edge_body,
        grid=(grid,),
        in_specs=in_specs,
        out_specs=[
            pl.BlockSpec((BR, 128), lambda i: (i, 0)),
            pl.BlockSpec((2, BR, 128), lambda i: (0, i, 0)),
            pl.BlockSpec((2, BR, 128), lambda i: (0, i, 0)),
        ],
        out_shape=[
            jax.ShapeDtypeStruct((E_PROWS, 128), jnp.float32),
            jax.ShapeDtypeStruct((2, E_PROWS, 128), jnp.float32),
            jax.ShapeDtypeStruct((2, E_PROWS, 128), jnp.float32),
        ],
    )(gi, gj, ea, *ws)


BRN = 1024
N_PROWS = N_NODES // 8


def _node_body(mp, mf, wp0, wp1, wf0, wf1, b0, w1, b1, out):
    f32 = jnp.float32
    h = jax.nn.relu(
        jnp.dot(mp[0], wp0[...], preferred_element_type=f32)
        + jnp.dot(mp[1], wp1[...], preferred_element_type=f32)
        + jnp.dot(mf[0], wf0[...], preferred_element_type=f32)
        + jnp.dot(mf[1], wf1[...], preferred_element_type=f32)
        + b0[...])
    out[...] = jnp.dot(h, w1[...], preferred_element_type=f32) + b1[...]


@jax.jit
def _tc_node(mp, mf, *ws):
    grid = pl.cdiv(N_PROWS, BRN)
    in_specs = [
        pl.BlockSpec((2, BRN, 128), lambda i: (0, i, 0)),
        pl.BlockSpec((2, BRN, 128), lambda i: (0, i, 0)),
    ] + [_wspec(w.shape) for w in ws]
    return pl.pallas_call(
        _node_body,
        grid=(grid,),
        in_specs=in_specs,
        out_specs=pl.BlockSpec((BRN, 128), lambda i: (i, 0)),
        out_shape=jax.ShapeDtypeStruct((N_PROWS, 128), jnp.float32),
    )(mp, mf, *ws)


def _bd8(w):
    return jnp.kron(jnp.eye(8, dtype=jnp.float32), w)


def _bt8(b):
    return jnp.tile(b, 8).reshape(1, -1)


def kernel(x, edge_index, edge_attr, initial_x,
           eu_w0, eu_b0, eu_w1, eu_b1, eu_w2, eu_b2,
           pm_w0, pm_b0, pm_w1, pm_b1,
           fm_w0, fm_b0, fm_w1, fm_b1,
           cf_w0, cf_b0, cf_w1, cf_b1):
    rows = edge_index[0]
    cols = edge_index[1]
    pad = E_PAD - N_EDGES
    zpad = jnp.zeros((pad,), jnp.int32)
    tpad = jnp.full((pad,), N_NODES, jnp.int32)
    cols_g = jnp.concatenate([cols, zpad]).reshape(E_ROWS, 128)
    rows_g = jnp.concatenate([rows, zpad]).reshape(E_ROWS, 128)
    cols_s = jnp.concatenate([cols, tpad]).reshape(E_ROWS, 128)
    rows_s = jnp.concatenate([rows, tpad]).reshape(E_ROWS, 128)

    z16 = jnp.zeros((ND, 2 * ND), jnp.float32)
    edge_ws = (
        _bd8(jnp.concatenate([eu_w0[:ND], z16], axis=0)),
        _bd8(jnp.concatenate([eu_w0[ND:2 * ND], z16], axis=0)),
        _bd8(eu_w0[2 * ND:]),
        _bt8(eu_b0),
        _bd8(eu_w1), _bt8(eu_b1),
        _bd8(eu_w2), _bt8(eu_b2),
        _bd8(jnp.concatenate([fm_w0[:ND], fm_w0[2 * ND:]], axis=0)),
        _bd8(fm_w0[ND:2 * ND]),
        _bt8(fm_b0),
        _bd8(fm_w1[:, :ED]), _bt8(fm_b1[:ED]),
        _bd8(fm_w1[:, ED:]), _bt8(fm_b1[ED:]),
        _bd8(jnp.concatenate([pm_w0[:ND], pm_w0[2 * ND:]], axis=0)),
        _bd8(pm_w0[ND:2 * ND]),
        _bt8(pm_b0),
        _bd8(pm_w1[:, :ED]), _bt8(pm_b1[:ED]),
        _bd8(pm_w1[:, ED:]), _bt8(pm_b1[ED:]),
    )
    node_ws = (
        _bd8(cf_w0[:ND]), _bd8(cf_w0[ND:2 * ND]),
        _bd8(cf_w0[2 * ND:3 * ND]), _bd8(cf_w0[3 * ND:]),
        _bt8(cf_b0),
        _bd8(cf_w1), _bt8(cf_b1),
    )

    ea_p = jnp.concatenate(
        [edge_attr, jnp.zeros((pad, ED), jnp.float32)]).reshape(E_PROWS, 128)
    for _ in range(3):
        comb = jnp.concatenate([x, initial_x], axis=1)
        gi, gj = _sc_gather(comb, cols_g, rows_g)
        gi_p = gi.reshape(E_PROWS, 256)
        gj_p = gj.reshape(E_PROWS, 256)
        ue_p, past_p, fut_p = _tc_edge(gi_p, gj_p, ea_p, *edge_ws)
        mp = _sc_scatter(past_p.reshape(2, E_PAD, ED), cols_s)
        mf = _sc_scatter(fut_p.reshape(2, E_PAD, ED), rows_s)
        xp = _tc_node(mp.reshape(2, N_PROWS, 128), mf.reshape(2, N_PROWS, 128),
                      *node_ws)
        x = xp.reshape(N_NODES, ND)
        ea_p = ue_p
    return x, ea_p.reshape(E_PAD, ED)[:N_EDGES]


# speedup vs baseline: 7.6246x; 1.2613x over previous
import functools

import jax
import jax.numpy as jnp
from jax import lax
from jax.experimental import pallas as pl
from jax.experimental.pallas import tpu as pltpu
from jax.experimental.pallas import tpu_sc as plsc

N_NODES = 100000
N_EDGES = 1600000
ND = 16
ED = 16
MD = 32

NC = 2
NS = 16
NW = NC * NS

E_PAD = 1605632
E_ROWS = E_PAD // 128

G_GRP = 49
G_EPW = E_PAD // NW


def _gather_body(table, colsr, rowsr, gi, gj,
                 idx0, idx1, buf0, buf1, si0, si1, sg, sw0, sw1):
    c = lax.axis_index("c")
    s = lax.axis_index("s")
    wid = s * NC + c
    idx_v = (idx0, idx1)
    buf_v = (buf0, buf1)
    si = (si0, si1)
    sw = (sw0, sw1)

    def do(idx_hbm, out_hbm):
        def fire_idx(g, sl):
            brow = wid * (G_EPW // 128) + g * 8
            pltpu.async_copy(idx_hbm.at[pl.ds(brow, 8)], idx_v[sl], si[sl])

        def slot_body(g, sl, k):
            pltpu.make_async_copy(idx_hbm.at[pl.ds(0, 8)], idx_v[sl], si[sl]).wait()

            @pl.when(k > 0)
            def _():
                pltpu.make_async_copy(
                    buf_v[sl], out_hbm.at[pl.ds(0, 1024)], sw[sl]).wait()

            descs = []
            for j in range(8):
                descs.append(pltpu.async_copy(
                    table.at[idx_v[sl].at[j]],
                    buf_v[sl].at[pl.ds(j * 128, 128)],
                    sg,
                ))
            for d in descs:
                d.wait()
            b = wid * G_EPW + g * 1024
            pltpu.async_copy(buf_v[sl], out_hbm.at[pl.ds(b, 1024)], sw[sl])

            @pl.when(g + 2 < G_GRP)
            def _():
                fire_idx(g + 2, sl)

        fire_idx(0, 0)
        fire_idx(1, 1)

        def pair(k, _):
            slot_body(2 * k, 0, k)
            slot_body(2 * k + 1, 1, k)
            return 0

        lax.fori_loop(0, G_GRP // 2, pair, 0)
        slot_body(G_GRP - 1, 0, 1)
        pltpu.make_async_copy(buf_v[1], out_hbm.at[pl.ds(0, 1024)], sw[1]).wait()
        pltpu.make_async_copy(buf_v[0], out_hbm.at[pl.ds(0, 1024)], sw[0]).wait()

    do(colsr, gi)
    do(rowsr, gj)


@jax.jit
def _sc_gather(table, cols2, rows2):
    return pl.kernel(
        _gather_body,
        out_type=[
            jax.ShapeDtypeStruct((E_PAD, ND), jnp.float32),
            jax.ShapeDtypeStruct((E_PAD, ND), jnp.float32),
        ],
        mesh=plsc.VectorSubcoreMesh(core_axis_name="c", subcore_axis_name="s"),
        scratch_types=[
            pltpu.VMEM((8, 128), jnp.int32),
            pltpu.VMEM((8, 128), jnp.int32),
            pltpu.VMEM((1024, ND), jnp.float32),
            pltpu.VMEM((1024, ND), jnp.float32),
            pltpu.SemaphoreType.DMA,
            pltpu.SemaphoreType.DMA,
            pltpu.SemaphoreType.DMA,
            pltpu.SemaphoreType.DMA,
            pltpu.SemaphoreType.DMA,
        ],
        compiler_params=pltpu.CompilerParams(use_tc_tiling_on_sc=False),
    )(table, cols2, rows2)


S_B = 512
S_NJ = S_B // 128
S_EPT = E_PAD // NS
S_GRP = S_EPT // S_B
ACC_ROWS = N_NODES + 8
NPT = N_NODES // NS
ZB = 125


def _scatter_body(msgs, idxs, out, acc,
                  idxv0, idxv1, mbuf0, mbuf1, zbuf, sd0, sd1, sa0, sa1, sz):
    c = lax.axis_index("c")
    s = lax.axis_index("s")
    idx_v = (idxv0, idxv1)
    mbuf = (mbuf0, mbuf1)
    sd = (sd0, sd1)
    sa = (sa0, sa1)

    def zrow(i, _):
        zbuf[i, :] = jnp.zeros((16,), jnp.float32)
        return 0

    lax.fori_loop(0, ZB, zrow, 0)
    r0 = s * NPT
    for k in range(NPT // ZB):
        pltpu.async_copy(zbuf, acc.at[pl.ds(r0 + k * ZB, ZB)], sz)
    for k in range(NPT // ZB):
        pltpu.make_async_copy(zbuf, acc.at[pl.ds(r0, ZB)], sz).wait()

    plsc.subcore_barrier()

    def fire_loads(g, sl):
        b = s * S_EPT + g * S_B
        brow = s * (S_EPT // 128) + g * S_NJ
        pltpu.async_copy(idxs.at[pl.ds(brow, S_NJ)], idx_v[sl], sd[sl])
        pltpu.async_copy(msgs.at[c, pl.ds(b, S_B)], mbuf[sl], sd[sl])

    def wait_loads(sl):
        pltpu.make_async_copy(idxs.at[pl.ds(0, S_NJ)], idx_v[sl], sd[sl]).wait()
        pltpu.make_async_copy(
            msgs.at[c, pl.ds(0, S_B)], mbuf[sl], sd[sl]).wait()

    def fire_adds(sl):
        descs = []
        for j in range(S_NJ):
            descs.append(pltpu.async_copy(
                mbuf[sl].at[pl.ds(j * 128, 128)],
                acc.at[idx_v[sl].at[j]],
                sa[sl],
                add=True,
            ))
        return descs

    fire_loads(0, 0)
    fire_loads(1, 1)

    def pair(k, _):
        g0 = 2 * k
        wait_loads(0)
        d0 = fire_adds(0)
        wait_loads(1)
        d1 = fire_adds(1)
        for d in d0:
            d.wait()

        @pl.when(g0 + 2 < S_GRP)
        def _():
            fire_loads(g0 + 2, 0)

        for d in d1:
            d.wait()

        @pl.when(g0 + 3 < S_GRP)
        def _():
            fire_loads(g0 + 3, 1)

        return 0

    lax.fori_loop(0, S_GRP // 2, pair, 0)
    plsc.subcore_barrier()
    pltpu.sync_copy(acc.at[pl.ds(r0, NPT)], out.at[c, pl.ds(r0, NPT)])


@jax.jit
def _sc_scatter(msgs, idxs2):
    return pl.kernel(
        _scatter_body,
        out_type=jax.ShapeDtypeStruct((2, N_NODES, ED), jnp.float32),
        mesh=plsc.VectorSubcoreMesh(core_axis_name="c", subcore_axis_name="s"),
        scratch_types=[
            pltpu.VMEM_SHARED((ACC_ROWS, ED), jnp.float32),
            pltpu.VMEM((S_NJ, 128), jnp.int32),
            pltpu.VMEM((S_NJ, 128), jnp.int32),
            pltpu.VMEM((S_B, ED), jnp.float32),
            pltpu.VMEM((S_B, ED), jnp.float32),
            pltpu.VMEM((ZB, ED), jnp.float32),
            pltpu.SemaphoreType.DMA,
            pltpu.SemaphoreType.DMA,
            pltpu.SemaphoreType.DMA,
            pltpu.SemaphoreType.DMA,
            pltpu.SemaphoreType.DMA,
        ],
        compiler_params=pltpu.CompilerParams(use_tc_tiling_on_sc=False),
    )(msgs, idxs2)


BR = 512
E_PROWS = E_PAD // 8
E_UROWS = N_EDGES // 8


def _edge_body(xi, xj, ii, ij, ea, wxi, wxj, wea, b0, w1, b1, w2, b2,
               fxi, fwu, fii, fb0, fw1a, fb1a, fw1b, fb1b,
               pxj, pwu, pij, pb0, pw1a, pb1a, pw1b, pb1b,
               ue, past2, fut2):
    f32 = jnp.float32
    XI = xi[...]
    XJ = xj[...]
    h = jax.nn.relu(
        jnp.dot(XI, wxi[...], preferred_element_type=f32)
        + jnp.dot(XJ, wxj[...], preferred_element_type=f32)
        + jnp.dot(ea[...], wea[...], preferred_element_type=f32)
        + b0[...])
    h = jax.nn.relu(jnp.dot(h, w1[...], preferred_element_type=f32) + b1[...])
    u = jnp.dot(h, w2[...], preferred_element_type=f32) + b2[...]
    ue[...] = u
    fh = jax.nn.relu(
        jnp.dot(XI, fxi[...], preferred_element_type=f32)
        + jnp.dot(u, fwu[...], preferred_element_type=f32)
        + jnp.dot(ii[...], fii[...], preferred_element_type=f32)
        + fb0[...])
    fut2[0] = jnp.dot(fh, fw1a[...], preferred_element_type=f32) + fb1a[...]
    fut2[1] = jnp.dot(fh, fw1b[...], preferred_element_type=f32) + fb1b[...]
    ph = jax.nn.relu(
        jnp.dot(XJ, pxj[...], preferred_element_type=f32)
        + jnp.dot(u, pwu[...], preferred_element_type=f32)
        + jnp.dot(ij[...], pij[...], preferred_element_type=f32)
        + pb0[...])
    past2[0] = jnp.dot(ph, pw1a[...], preferred_element_type=f32) + pb1a[...]
    past2[1] = jnp.dot(ph, pw1b[...], preferred_element_type=f32) + pb1b[...]


def _wspec(shape):
    return pl.BlockSpec(shape, lambda i: (0,) * len(shape))


@jax.jit
def _tc_edge(xi, xj, ii, ij, ea, *ws):
    grid = E_PROWS // BR
    in_specs = [
        pl.BlockSpec((BR, 128), lambda i: (i, 0)),
        pl.BlockSpec((BR, 128), lambda i: (i, 0)),
        pl.BlockSpec((BR, 128), lambda i: (i, 0)),
        pl.BlockSpec((BR, 128), lambda i: (i, 0)),
        pl.BlockSpec((BR, 128), lambda i: (i, 0)),
    ] + [_wspec(w.shape) for w in ws]
    return pl.pallas_call(
        _edge_body,
        grid=(grid,),
        in_specs=in_specs,
        out_specs=[
            pl.BlockSpec((BR, 128), lambda i: (i, 0)),
            pl.BlockSpec((2, BR, 128), lambda i: (0, i, 0)),
            pl.BlockSpec((2, BR, 128), lambda i: (0, i, 0)),
        ],
        out_shape=[
            jax.ShapeDtypeStruct((E_PROWS, 128), jnp.float32),
            jax.ShapeDtypeStruct((2, E_PROWS, 128), jnp.float32),
            jax.ShapeDtypeStruct((2, E_PROWS, 128), jnp.float32),
        ],
    )(xi, xj, ii, ij, ea, *ws)


BRN = 1024
N_PROWS = N_NODES // 8


def _node_body(mp, mf, wp0, wp1, wf0, wf1, b0, w1, b1, out):
    f32 = jnp.float32
    h = jax.nn.relu(
        jnp.dot(mp[0], wp0[...], preferred_element_type=f32)
        + jnp.dot(mp[1], wp1[...], preferred_element_type=f32)
        + jnp.dot(mf[0], wf0[...], preferred_element_type=f32)
        + jnp.dot(mf[1], wf1[...], preferred_element_type=f32)
        + b0[...])
    out[...] = jnp.dot(h, w1[...], preferred_element_type=f32) + b1[...]


@jax.jit
def _tc_node(mp, mf, *ws):
    grid = pl.cdiv(N_PROWS, BRN)
    in_specs = [
        pl.BlockSpec((2, BRN, 128), lambda i: (0, i, 0)),
        pl.BlockSpec((2, BRN, 128), lambda i: (0, i, 0)),
    ] + [_wspec(w.shape) for w in ws]
    return pl.pallas_call(
        _node_body,
        grid=(grid,),
        in_specs=in_specs,
        out_specs=pl.BlockSpec((BRN, 128), lambda i: (i, 0)),
        out_shape=jax.ShapeDtypeStruct((N_PROWS, 128), jnp.float32),
    )(mp, mf, *ws)


def _bd8(w):
    return jnp.kron(jnp.eye(8, dtype=jnp.float32), w)


def _bt8(b):
    return jnp.tile(b, 8).reshape(1, -1)


def kernel(x, edge_index, edge_attr, initial_x,
           eu_w0, eu_b0, eu_w1, eu_b1, eu_w2, eu_b2,
           pm_w0, pm_b0, pm_w1, pm_b1,
           fm_w0, fm_b0, fm_w1, fm_b1,
           cf_w0, cf_b0, cf_w1, cf_b1):
    rows = edge_index[0]
    cols = edge_index[1]
    pad = E_PAD - N_EDGES
    zpad = jnp.zeros((pad,), jnp.int32)
    tpad = jnp.full((pad,), N_NODES, jnp.int32)
    cols_g = jnp.concatenate([cols, zpad]).reshape(E_ROWS, 128)
    rows_g = jnp.concatenate([rows, zpad]).reshape(E_ROWS, 128)
    cols_s = jnp.concatenate([cols, tpad]).reshape(E_ROWS, 128)
    rows_s = jnp.concatenate([rows, tpad]).reshape(E_ROWS, 128)

    edge_ws = (
        _bd8(eu_w0[:ND]),
        _bd8(eu_w0[ND:2 * ND]),
        _bd8(eu_w0[2 * ND:]),
        _bt8(eu_b0),
        _bd8(eu_w1), _bt8(eu_b1),
        _bd8(eu_w2), _bt8(eu_b2),
        _bd8(fm_w0[:ND]),
        _bd8(fm_w0[ND:2 * ND]),
        _bd8(fm_w0[2 * ND:]),
        _bt8(fm_b0),
        _bd8(fm_w1[:, :ED]), _bt8(fm_b1[:ED]),
        _bd8(fm_w1[:, ED:]), _bt8(fm_b1[ED:]),
        _bd8(pm_w0[:ND]),
        _bd8(pm_w0[ND:2 * ND]),
        _bd8(pm_w0[2 * ND:]),
        _bt8(pm_b0),
        _bd8(pm_w1[:, :ED]), _bt8(pm_b1[:ED]),
        _bd8(pm_w1[:, ED:]), _bt8(pm_b1[ED:]),
    )
    node_ws = (
        _bd8(cf_w0[:ND]), _bd8(cf_w0[ND:2 * ND]),
        _bd8(cf_w0[2 * ND:3 * ND]), _bd8(cf_w0[3 * ND:]),
        _bt8(cf_b0),
        _bd8(cf_w1), _bt8(cf_b1),
    )

    ea_p = jnp.concatenate(
        [edge_attr, jnp.zeros((pad, ED), jnp.float32)]).reshape(E_PROWS, 128)
    ii, ij = _sc_gather(initial_x, cols_g, rows_g)
    ii_p = ii.reshape(E_PROWS, 128)
    ij_p = ij.reshape(E_PROWS, 128)
    for _ in range(3):
        xi, xj = _sc_gather(x, cols_g, rows_g)
        ue_p, past_p, fut_p = _tc_edge(
            xi.reshape(E_PROWS, 128), xj.reshape(E_PROWS, 128),
            ii_p, ij_p, ea_p, *edge_ws)
        mp = _sc_scatter(past_p.reshape(2, E_PAD, ED), cols_s)
        mf = _sc_scatter(fut_p.reshape(2, E_PAD, ED), rows_s)
        xp = _tc_node(mp.reshape(2, N_PROWS, 128), mf.reshape(2, N_PROWS, 128),
                      *node_ws)
        x = xp.reshape(N_NODES, ND)
        ea_p = ue_p
    return x, ea_p.reshape(E_PAD, ED)[:N_EDGES]


# speedup vs baseline: 7.6626x; 1.0050x over previous
import functools

import jax
import jax.numpy as jnp
from jax import lax
from jax.experimental import pallas as pl
from jax.experimental.pallas import tpu as pltpu
from jax.experimental.pallas import tpu_sc as plsc

N_NODES = 100000
N_EDGES = 1600000
ND = 16
ED = 16
MD = 32

NC = 2
NS = 16
NW = NC * NS

E_PAD = 1605632
E_ROWS = E_PAD // 128

G_GRP = 49
G_EPW = E_PAD // NW


def _gather_body(table, colsr, rowsr, gi, gj,
                 idx0, idx1, buf0, buf1, si0, si1, sg0, sg1, sw0, sw1):
    c = lax.axis_index("c")
    s = lax.axis_index("s")
    wid = s * NC + c
    idx_v = (idx0, idx1)
    buf_v = (buf0, buf1)
    si = (si0, si1)
    sg = (sg0, sg1)
    sw = (sw0, sw1)

    def do(idx_hbm, out_hbm):
        def fire_idx(g, sl):
            brow = wid * (G_EPW // 128) + g * 8
            pltpu.async_copy(idx_hbm.at[pl.ds(brow, 8)], idx_v[sl], si[sl])

        def fire_gathers(sl, k):
            pltpu.make_async_copy(idx_hbm.at[pl.ds(0, 8)], idx_v[sl], si[sl]).wait()

            @pl.when(k > 0)
            def _():
                pltpu.make_async_copy(
                    buf_v[sl], out_hbm.at[pl.ds(0, 1024)], sw[sl]).wait()

            descs = []
            for j in range(8):
                descs.append(pltpu.async_copy(
                    table.at[idx_v[sl].at[j]],
                    buf_v[sl].at[pl.ds(j * 128, 128)],
                    sg[sl],
                ))
            return descs

        def drain_slot(g, sl, descs):
            for d in descs:
                d.wait()
            b = wid * G_EPW + g * 1024
            pltpu.async_copy(buf_v[sl], out_hbm.at[pl.ds(b, 1024)], sw[sl])

            @pl.when(g + 2 < G_GRP)
            def _():
                fire_idx(g + 2, sl)

        fire_idx(0, 0)
        fire_idx(1, 1)

        def pair(k, _):
            d0 = fire_gathers(0, k)
            d1 = fire_gathers(1, k)
            drain_slot(2 * k, 0, d0)
            drain_slot(2 * k + 1, 1, d1)
            return 0

        lax.fori_loop(0, G_GRP // 2, pair, 0)
        d0 = fire_gathers(0, 1)
        drain_slot(G_GRP - 1, 0, d0)
        pltpu.make_async_copy(buf_v[1], out_hbm.at[pl.ds(0, 1024)], sw[1]).wait()
        pltpu.make_async_copy(buf_v[0], out_hbm.at[pl.ds(0, 1024)], sw[0]).wait()

    do(colsr, gi)
    do(rowsr, gj)


@jax.jit
def _sc_gather(table, cols2, rows2):
    return pl.kernel(
        _gather_body,
        out_type=[
            jax.ShapeDtypeStruct((E_PAD, ND), jnp.float32),
            jax.ShapeDtypeStruct((E_PAD, ND), jnp.float32),
        ],
        mesh=plsc.VectorSubcoreMesh(core_axis_name="c", subcore_axis_name="s"),
        scratch_types=[
            pltpu.VMEM((8, 128), jnp.int32),
            pltpu.VMEM((8, 128), jnp.int32),
            pltpu.VMEM((1024, ND), jnp.float32),
            pltpu.VMEM((1024, ND), jnp.float32),
            pltpu.SemaphoreType.DMA,
            pltpu.SemaphoreType.DMA,
            pltpu.SemaphoreType.DMA,
            pltpu.SemaphoreType.DMA,
            pltpu.SemaphoreType.DMA,
            pltpu.SemaphoreType.DMA,
        ],
        compiler_params=pltpu.CompilerParams(use_tc_tiling_on_sc=False),
    )(table, cols2, rows2)


S_B = 512
S_NJ = S_B // 128
S_EPT = E_PAD // NS
S_GRP = S_EPT // S_B
ACC_ROWS = N_NODES + 8
NPT = N_NODES // NS
ZB = 125


def _scatter_body(msgs, idxs, out, acc,
                  idxv0, idxv1, mbuf0, mbuf1, zbuf, sd0, sd1, sa0, sa1, sz):
    c = lax.axis_index("c")
    s = lax.axis_index("s")
    idx_v = (idxv0, idxv1)
    mbuf = (mbuf0, mbuf1)
    sd = (sd0, sd1)
    sa = (sa0, sa1)

    def zrow(i, _):
        zbuf[i, :] = jnp.zeros((16,), jnp.float32)
        return 0

    lax.fori_loop(0, ZB, zrow, 0)
    r0 = s * NPT
    for k in range(NPT // ZB):
        pltpu.async_copy(zbuf, acc.at[pl.ds(r0 + k * ZB, ZB)], sz)
    for k in range(NPT // ZB):
        pltpu.make_async_copy(zbuf, acc.at[pl.ds(r0, ZB)], sz).wait()

    plsc.subcore_barrier()

    def fire_loads(g, sl):
        b = s * S_EPT + g * S_B
        brow = s * (S_EPT // 128) + g * S_NJ
        pltpu.async_copy(idxs.at[pl.ds(brow, S_NJ)], idx_v[sl], sd[sl])
        pltpu.async_copy(msgs.at[c, pl.ds(b, S_B)], mbuf[sl], sd[sl])

    def wait_loads(sl):
        pltpu.make_async_copy(idxs.at[pl.ds(0, S_NJ)], idx_v[sl], sd[sl]).wait()
        pltpu.make_async_copy(
            msgs.at[c, pl.ds(0, S_B)], mbuf[sl], sd[sl]).wait()

    def fire_adds(sl):
        descs = []
        for j in range(S_NJ):
            descs.append(pltpu.async_copy(
                mbuf[sl].at[pl.ds(j * 128, 128)],
                acc.at[idx_v[sl].at[j]],
                sa[sl],
                add=True,
            ))
        return descs

    fire_loads(0, 0)
    fire_loads(1, 1)

    def pair(k, _):
        g0 = 2 * k
        wait_loads(0)
        d0 = fire_adds(0)
        wait_loads(1)
        d1 = fire_adds(1)
        for d in d0:
            d.wait()

        @pl.when(g0 + 2 < S_GRP)
        def _():
            fire_loads(g0 + 2, 0)

        for d in d1:
            d.wait()

        @pl.when(g0 + 3 < S_GRP)
        def _():
            fire_loads(g0 + 3, 1)

        return 0

    lax.fori_loop(0, S_GRP // 2, pair, 0)
    plsc.subcore_barrier()
    pltpu.sync_copy(acc.at[pl.ds(r0, NPT)], out.at[c, pl.ds(r0, NPT)])


@jax.jit
def _sc_scatter(msgs, idxs2):
    return pl.kernel(
        _scatter_body,
        out_type=jax.ShapeDtypeStruct((2, N_NODES, ED), jnp.float32),
        mesh=plsc.VectorSubcoreMesh(core_axis_name="c", subcore_axis_name="s"),
        scratch_types=[
            pltpu.VMEM_SHARED((ACC_ROWS, ED), jnp.float32),
            pltpu.VMEM((S_NJ, 128), jnp.int32),
            pltpu.VMEM((S_NJ, 128), jnp.int32),
            pltpu.VMEM((S_B, ED), jnp.float32),
            pltpu.VMEM((S_B, ED), jnp.float32),
            pltpu.VMEM((ZB, ED), jnp.float32),
            pltpu.SemaphoreType.DMA,
            pltpu.SemaphoreType.DMA,
            pltpu.SemaphoreType.DMA,
            pltpu.SemaphoreType.DMA,
            pltpu.SemaphoreType.DMA,
        ],
        compiler_params=pltpu.CompilerParams(use_tc_tiling_on_sc=False),
    )(msgs, idxs2)


BR = 512
E_PROWS = E_PAD // 8
E_UROWS = N_EDGES // 8


def _edge_body(xi, xj, ii, ij, ea, wxi, wxj, wea, b0, w1, b1, w2, b2,
               fxi, fwu, fii, fb0, fw1a, fb1a, fw1b, fb1b,
               pxj, pwu, pij, pb0, pw1a, pb1a, pw1b, pb1b,
               ue, past2, fut2):
    f32 = jnp.float32
    XI = xi[...]
    XJ = xj[...]
    h = jax.nn.relu(
        jnp.dot(XI, wxi[...], preferred_element_type=f32)
        + jnp.dot(XJ, wxj[...], preferred_element_type=f32)
        + jnp.dot(ea[...], wea[...], preferred_element_type=f32)
        + b0[...])
    h = jax.nn.relu(jnp.dot(h, w1[...], preferred_element_type=f32) + b1[...])
    u = jnp.dot(h, w2[...], preferred_element_type=f32) + b2[...]
    ue[...] = u
    fh = jax.nn.relu(
        jnp.dot(XI, fxi[...], preferred_element_type=f32)
        + jnp.dot(u, fwu[...], preferred_element_type=f32)
        + jnp.dot(ii[...], fii[...], preferred_element_type=f32)
        + fb0[...])
    fut2[0] = jnp.dot(fh, fw1a[...], preferred_element_type=f32) + fb1a[...]
    fut2[1] = jnp.dot(fh, fw1b[...], preferred_element_type=f32) + fb1b[...]
    ph = jax.nn.relu(
        jnp.dot(XJ, pxj[...], preferred_element_type=f32)
        + jnp.dot(u, pwu[...], preferred_element_type=f32)
        + jnp.dot(ij[...], pij[...], preferred_element_type=f32)
        + pb0[...])
    past2[0] = jnp.dot(ph, pw1a[...], preferred_element_type=f32) + pb1a[...]
    past2[1] = jnp.dot(ph, pw1b[...], preferred_element_type=f32) + pb1b[...]


def _wspec(shape):
    return pl.BlockSpec(shape, lambda i: (0,) * len(shape))


@jax.jit
def _tc_edge(xi, xj, ii, ij, ea, *ws):
    grid = E_PROWS // BR
    in_specs = [
        pl.BlockSpec((BR, 128), lambda i: (i, 0)),
        pl.BlockSpec((BR, 128), lambda i: (i, 0)),
        pl.BlockSpec((BR, 128), lambda i: (i, 0)),
        pl.BlockSpec((BR, 128), lambda i: (i, 0)),
        pl.BlockSpec((BR, 128), lambda i: (i, 0)),
    ] + [_wspec(w.shape) for w in ws]
    return pl.pallas_call(
        _edge_body,
        grid=(grid,),
        in_specs=in_specs,
        out_specs=[
            pl.BlockSpec((BR, 128), lambda i: (i, 0)),
            pl.BlockSpec((2, BR, 128), lambda i: (0, i, 0)),
            pl.BlockSpec((2, BR, 128), lambda i: (0, i, 0)),
        ],
        out_shape=[
            jax.ShapeDtypeStruct((E_PROWS, 128), jnp.float32),
            jax.ShapeDtypeStruct((2, E_PROWS, 128), jnp.float32),
            jax.ShapeDtypeStruct((2, E_PROWS, 128), jnp.float32),
        ],
    )(xi, xj, ii, ij, ea, *ws)


BRN = 1024
N_PROWS = N_NODES // 8


def _node_body(mp, mf, wp0, wp1, wf0, wf1, b0, w1, b1, out):
    f32 = jnp.float32
    h = jax.nn.relu(
        jnp.dot(mp[0], wp0[...], preferred_element_type=f32)
        + jnp.dot(mp[1], wp1[...], preferred_element_type=f32)
        + jnp.dot(mf[0], wf0[...], preferred_element_type=f32)
        + jnp.dot(mf[1], wf1[...], preferred_element_type=f32)
        + b0[...])
    out[...] = jnp.dot(h, w1[...], preferred_element_type=f32) + b1[...]


@jax.jit
def _tc_node(mp, mf, *ws):
    grid = pl.cdiv(N_PROWS, BRN)
    in_specs = [
        pl.BlockSpec((2, BRN, 128), lambda i: (0, i, 0)),
        pl.BlockSpec((2, BRN, 128), lambda i: (0, i, 0)),
    ] + [_wspec(w.shape) for w in ws]
    return pl.pallas_call(
        _node_body,
        grid=(grid,),
        in_specs=in_specs,
        out_specs=pl.BlockSpec((BRN, 128), lambda i: (i, 0)),
        out_shape=jax.ShapeDtypeStruct((N_PROWS, 128), jnp.float32),
    )(mp, mf, *ws)


def _bd8(w):
    return jnp.kron(jnp.eye(8, dtype=jnp.float32), w)


def _bt8(b):
    return jnp.tile(b, 8).reshape(1, -1)


def kernel(x, edge_index, edge_attr, initial_x,
           eu_w0, eu_b0, eu_w1, eu_b1, eu_w2, eu_b2,
           pm_w0, pm_b0, pm_w1, pm_b1,
           fm_w0, fm_b0, fm_w1, fm_b1,
           cf_w0, cf_b0, cf_w1, cf_b1):
    rows = edge_index[0]
    cols = edge_index[1]
    pad = E_PAD - N_EDGES
    zpad = jnp.zeros((pad,), jnp.int32)
    tpad = jnp.full((pad,), N_NODES, jnp.int32)
    cols_g = jnp.concatenate([cols, zpad]).reshape(E_ROWS, 128)
    rows_g = jnp.concatenate([rows, zpad]).reshape(E_ROWS, 128)
    cols_s = jnp.concatenate([cols, tpad]).reshape(E_ROWS, 128)
    rows_s = jnp.concatenate([rows, tpad]).reshape(E_ROWS, 128)

    edge_ws = (
        _bd8(eu_w0[:ND]),
        _bd8(eu_w0[ND:2 * ND]),
        _bd8(eu_w0[2 * ND:]),
        _bt8(eu_b0),
        _bd8(eu_w1), _bt8(eu_b1),
        _bd8(eu_w2), _bt8(eu_b2),
        _bd8(fm_w0[:ND]),
        _bd8(fm_w0[ND:2 * ND]),
        _bd8(fm_w0[2 * ND:]),
        _bt8(fm_b0),
        _bd8(fm_w1[:, :ED]), _bt8(fm_b1[:ED]),
        _bd8(fm_w1[:, ED:]), _bt8(fm_b1[ED:]),
        _bd8(pm_w0[:ND]),
        _bd8(pm_w0[ND:2 * ND]),
        _bd8(pm_w0[2 * ND:]),
        _bt8(pm_b0),
        _bd8(pm_w1[:, :ED]), _bt8(pm_b1[:ED]),
        _bd8(pm_w1[:, ED:]), _bt8(pm_b1[ED:]),
    )
    node_ws = (
        _bd8(cf_w0[:ND]), _bd8(cf_w0[ND:2 * ND]),
        _bd8(cf_w0[2 * ND:3 * ND]), _bd8(cf_w0[3 * ND:]),
        _bt8(cf_b0),
        _bd8(cf_w1), _bt8(cf_b1),
    )

    ea_p = jnp.concatenate(
        [edge_attr, jnp.zeros((pad, ED), jnp.float32)]).reshape(E_PROWS, 128)
    ii, ij = _sc_gather(initial_x, cols_g, rows_g)
    ii_p = ii.reshape(E_PROWS, 128)
    ij_p = ij.reshape(E_PROWS, 128)
    for _ in range(3):
        xi, xj = _sc_gather(x, cols_g, rows_g)
        ue_p, past_p, fut_p = _tc_edge(
            xi.reshape(E_PROWS, 128), xj.reshape(E_PROWS, 128),
            ii_p, ij_p, ea_p, *edge_ws)
        mp = _sc_scatter(past_p.reshape(2, E_PAD, ED), cols_s)
        mf = _sc_scatter(fut_p.reshape(2, E_PAD, ED), rows_s)
        xp = _tc_node(mp.reshape(2, N_PROWS, 128), mf.reshape(2, N_PROWS, 128),
                      *node_ws)
        x = xp.reshape(N_NODES, ND)
        ea_p = ue_p
    return x, ea_p.reshape(E_PAD, ED)[:N_EDGES]
